# Initial kernel scaffold; baseline (speedup 1.0000x reference)
import jax
import jax.numpy as jnp
from jax.experimental import pallas as pl


def kernel(images, mu):
    raise NotImplementedError("write your pallas kernel here")


# speedup vs baseline: 34.5493x; 34.5493x over previous
import jax
import jax.numpy as jnp
from jax.experimental import pallas as pl


def _loss_kernel(x_ref, mu_ref, out_ref):
    x = x_ref[...]
    mu = mu_ref[...]
    dots = jnp.dot(x, mu, preferred_element_type=jnp.float32)
    mu_nsq = jnp.sum(mu * mu, axis=0, keepdims=True)
    score = mu_nsq - 2.0 * dots
    m = jnp.min(score, axis=1, keepdims=True)
    x_nsq = jnp.sum(x * x, axis=1, keepdims=True)
    out_ref[...] = (x_nsq + m) * (1.0 / x.shape[1])


def kernel(images, mu):
    B, G = images.shape
    out = pl.pallas_call(
        _loss_kernel,
        out_shape=jax.ShapeDtypeStruct((B, 1), jnp.float32),
    )(images, mu)
    return out[:, 0]


# speedup vs baseline: 50.6163x; 1.4650x over previous
import jax
import jax.numpy as jnp
from jax.experimental import pallas as pl


def _loss_kernel(x_ref, mu_ref, out_ref):
    x = x_ref[...]
    mu = mu_ref[...]
    dots = jnp.dot(x, mu, preferred_element_type=jnp.float32)
    mu_nsq = jnp.sum(mu * mu, axis=0, keepdims=True)
    score = mu_nsq - 2.0 * dots
    m = jnp.min(score, axis=1)
    x_nsq = jnp.sum(x * x, axis=1)
    out_ref[...] = (x_nsq + m) * (1.0 / x.shape[1])


def kernel(images, mu):
    B, G = images.shape
    return pl.pallas_call(
        _loss_kernel,
        out_shape=jax.ShapeDtypeStruct((B,), jnp.float32),
    )(images, mu)
